# baseline (speedup 1.0000x reference)
import functools

import jax
import jax.numpy as jnp
from jax import lax
from jax.experimental import pallas as pl
from jax.experimental.pallas import tpu as pltpu
from jax.experimental.pallas import tpu_sc as plsc

_TILE = 128
_K = 2


def _router_body(x_ref, wg_ref, idx_ref, w_ref):
    x = x_ref[...]
    wg = wg_ref[...]
    logits = lax.dot_general(x, wg, (((1,), (1,)), ((), ())),
                             preferred_element_type=jnp.float32)
    s, e = logits.shape
    eio = lax.broadcasted_iota(jnp.int32, (s, e), 1)
    m1 = jnp.max(logits, axis=1, keepdims=True)
    i1 = jnp.min(jnp.where(logits == m1, eio, e), axis=1, keepdims=True)
    l2 = jnp.where(eio == i1, -jnp.inf, logits)
    m2 = jnp.max(l2, axis=1, keepdims=True)
    i2 = jnp.min(jnp.where(l2 == m2, eio, e), axis=1, keepdims=True)
    w1 = 1.0 / (1.0 + jnp.exp(m2 - m1))
    idx_ref[...] = jnp.where(eio == 0, i1, jnp.where(eio == 1, i2, 0))
    w_ref[...] = jnp.where(eio == 0, w1, jnp.where(eio == 1, 1.0 - w1, 0.0))


def _run_router(x, gate_weight):
    s, _ = x.shape
    e = gate_weight.shape[0]
    return pl.pallas_call(
        _router_body,
        out_shape=(
            jax.ShapeDtypeStruct((s, e), jnp.int32),
            jax.ShapeDtypeStruct((s, e), jnp.float32),
        ),
    )(x, gate_weight)


def _make_sc_gather(n_rows, d):
    info = plsc.get_sparse_core_info()
    nw = info.num_cores * info.num_subcores
    per = n_rows // nw
    mesh = plsc.VectorSubcoreMesh(core_axis_name="c", subcore_axis_name="s")

    @functools.partial(
        pl.kernel,
        out_type=jax.ShapeDtypeStruct((n_rows, d), jnp.float32),
        mesh=mesh,
        scratch_types=[
            pltpu.VMEM((per,), jnp.int32),
            pltpu.VMEM((per, d), jnp.float32),
            pltpu.SemaphoreType.DMA,
        ],
    )
    def k(table_hbm, idx_hbm, out_hbm, idx_v, rows_v, sem):
        wid = lax.axis_index("s") * info.num_cores + lax.axis_index("c")
        base = wid * per
        pltpu.sync_copy(idx_hbm.at[pl.ds(base, per)], idx_v)
        pltpu.async_copy(table_hbm.at[idx_v], rows_v, sem).wait()
        pltpu.sync_copy(rows_v, out_hbm.at[pl.ds(base, per)])

    return k


def _make_sc_combine(n_tok, d):
    info = plsc.get_sparse_core_info()
    nw = info.num_cores * info.num_subcores
    per_t = n_tok // nw
    per_r = _K * per_t
    mesh = plsc.VectorSubcoreMesh(core_axis_name="c", subcore_axis_name="s")

    @functools.partial(
        pl.kernel,
        out_type=jax.ShapeDtypeStruct((n_tok, d), jnp.float32),
        mesh=mesh,
        scratch_types=[
            pltpu.VMEM((per_r,), jnp.int32),
            pltpu.VMEM((per_r, d), jnp.float32),
            pltpu.SemaphoreType.DMA,
        ],
    )
    def k(h_hbm, inv_hbm, out_hbm, idx_v, rows_v, sem):
        wid = lax.axis_index("s") * info.num_cores + lax.axis_index("c")
        pltpu.sync_copy(inv_hbm.at[pl.ds(wid * per_r, per_r)], idx_v)
        pltpu.async_copy(h_hbm.at[idx_v], rows_v, sem).wait()

        def body(i, carry):
            for c in range(d // info.num_lanes):
                sl = pl.ds(c * info.num_lanes, info.num_lanes)
                rows_v[i, sl] = rows_v[2 * i, sl] + rows_v[2 * i + 1, sl]
            return carry

        lax.fori_loop(0, per_t, body, 0)
        pltpu.sync_copy(rows_v.at[pl.ds(0, per_t)],
                        out_hbm.at[pl.ds(wid * per_t, per_t)])

    return k


def _gelu_exact(a):
    return 0.5 * a * (1.0 + lax.erf(a * 0.7071067811865476))


def _ffn_body(tile_ref, exp_ref, start_ref, end_ref,
              xg_ref, fc_ref, proj_ref, gate_ref, h_ref):
    p = pl.program_id(0)
    t = tile_ref[p]
    gid = t * _TILE + lax.broadcasted_iota(jnp.int32, (_TILE, 1), 0)
    mask = (gid >= start_ref[p]) & (gid < end_ref[p])
    x = jnp.where(mask, xg_ref[...], 0.0).astype(jnp.bfloat16)
    a = lax.dot_general(x, fc_ref[0], (((1,), (1,)), ((), ())),
                        preferred_element_type=jnp.float32)
    g = _gelu_exact(a).astype(jnp.bfloat16)
    h = lax.dot_general(g, proj_ref[0], (((1,), (1,)), ((), ())),
                        preferred_element_type=jnp.float32)
    h = h * gate_ref[...]
    first = (p == 0) | (t != tile_ref[jnp.maximum(p - 1, 0)])

    @pl.when(first)
    def _():
        h_ref[...] = h

    @pl.when(jnp.logical_not(first))
    def _():
        h_ref[...] += h


def _run_ffn(pair_tile, pair_exp, pair_start, pair_end,
             xg, c_fc_weight, c_proj_weight, gates2d):
    n_rows, d = xg.shape
    e, dff, _ = c_fc_weight.shape
    np_ = pair_tile.shape[0]
    grid_spec = pltpu.PrefetchScalarGridSpec(
        num_scalar_prefetch=4,
        grid=(np_,),
        in_specs=[
            pl.BlockSpec((_TILE, d), lambda p, tr, er, sr, nr: (tr[p], 0)),
            pl.BlockSpec((1, dff, d), lambda p, tr, er, sr, nr: (er[p], 0, 0)),
            pl.BlockSpec((1, d, dff), lambda p, tr, er, sr, nr: (er[p], 0, 0)),
            pl.BlockSpec((_TILE, 1), lambda p, tr, er, sr, nr: (tr[p], 0)),
        ],
        out_specs=pl.BlockSpec((_TILE, d), lambda p, tr, er, sr, nr: (tr[p], 0)),
    )
    return pl.pallas_call(
        _ffn_body,
        grid_spec=grid_spec,
        out_shape=jax.ShapeDtypeStruct((n_rows, d), jnp.float32),
        compiler_params=pltpu.CompilerParams(
            dimension_semantics=("arbitrary",)),
    )(pair_tile, pair_exp, pair_start, pair_end,
      xg, c_fc_weight, c_proj_weight, gates2d)


def kernel(hidden_states, gate_weight, c_fc_weight, c_proj_weight):
    b, s, d = hidden_states.shape
    e, dff, _ = c_fc_weight.shape
    x = hidden_states.reshape(-1, d)
    n_tok = x.shape[0]
    n_rows = n_tok * _K
    nt = n_rows // _TILE

    idx8, w8 = _run_router(x, gate_weight)
    sel_flat = idx8[:, :_K].reshape(-1)
    gates_flat = w8[:, :_K].reshape(-1)

    perm = jnp.argsort(sel_flat, stable=True).astype(jnp.int32)
    batch_index = perm // _K
    gates_sorted = gates_flat[perm]
    sel_sorted = sel_flat[perm]
    interior = jnp.searchsorted(
        sel_sorted, jnp.arange(1, e, dtype=sel_sorted.dtype),
        side="left").astype(jnp.int32)
    tile_starts = jnp.arange(nt, dtype=jnp.int32) * _TILE
    bounds = jnp.sort(jnp.concatenate([tile_starts, interior]))
    pair_tile = jnp.clip(bounds // _TILE, 0, nt - 1)
    pair_exp = jnp.clip(
        jnp.searchsorted(interior, bounds, side="right").astype(jnp.int32),
        0, e - 1)
    pair_start = bounds
    pair_end = jnp.concatenate(
        [bounds[1:], jnp.array([n_rows], dtype=jnp.int32)])
    inv = jnp.zeros((n_rows,), jnp.int32).at[perm].set(
        jnp.arange(n_rows, dtype=jnp.int32))

    xg = _make_sc_gather(n_rows, d)(x, batch_index)
    h = _run_ffn(pair_tile, pair_exp, pair_start, pair_end,
                 xg, c_fc_weight.astype(jnp.bfloat16),
                 c_proj_weight.astype(jnp.bfloat16),
                 gates_sorted.reshape(n_rows, 1))
    out = _make_sc_combine(n_tok, d)(h, inv)
    return out.reshape(b, s, d)


# speedup vs baseline: 1.1597x; 1.1597x over previous
import functools

import jax
import jax.numpy as jnp
from jax import lax
from jax.experimental import pallas as pl
from jax.experimental.pallas import tpu as pltpu
from jax.experimental.pallas import tpu_sc as plsc

_TILE = 128
_K = 2


def _router_body(x_ref, wg_ref, idx_ref, w_ref):
    x = x_ref[...]
    wg = wg_ref[...]
    logits = lax.dot_general(x, wg, (((1,), (1,)), ((), ())),
                             preferred_element_type=jnp.float32)
    s, e = logits.shape
    eio = lax.broadcasted_iota(jnp.int32, (s, e), 1)
    m1 = jnp.max(logits, axis=1, keepdims=True)
    i1 = jnp.min(jnp.where(logits == m1, eio, e), axis=1, keepdims=True)
    l2 = jnp.where(eio == i1, -jnp.inf, logits)
    m2 = jnp.max(l2, axis=1, keepdims=True)
    i2 = jnp.min(jnp.where(l2 == m2, eio, e), axis=1, keepdims=True)
    w1 = 1.0 / (1.0 + jnp.exp(m2 - m1))
    idx_ref[...] = jnp.where(eio == 0, i1, jnp.where(eio == 1, i2, 0))
    w_ref[...] = jnp.where(eio == 0, w1, jnp.where(eio == 1, 1.0 - w1, 0.0))


def _run_router(x, gate_weight):
    s, _ = x.shape
    e = gate_weight.shape[0]
    return pl.pallas_call(
        _router_body,
        out_shape=(
            jax.ShapeDtypeStruct((s, e), jnp.int32),
            jax.ShapeDtypeStruct((s, e), jnp.float32),
        ),
    )(x, gate_weight)


def _make_sc_gather(n_rows, d):
    info = plsc.get_sparse_core_info()
    nw = info.num_cores * info.num_subcores
    per = n_rows // nw
    mesh = plsc.VectorSubcoreMesh(core_axis_name="c", subcore_axis_name="s")

    @functools.partial(
        pl.kernel,
        out_type=jax.ShapeDtypeStruct((n_rows, d), jnp.float32),
        mesh=mesh,
        scratch_types=[
            pltpu.VMEM((per,), jnp.int32),
            pltpu.VMEM((per, d), jnp.float32),
            pltpu.SemaphoreType.DMA,
        ],
    )
    def k(table_hbm, idx_hbm, out_hbm, idx_v, rows_v, sem):
        wid = lax.axis_index("s") * info.num_cores + lax.axis_index("c")
        base = wid * per
        pltpu.sync_copy(idx_hbm.at[pl.ds(base, per)], idx_v)
        pltpu.async_copy(table_hbm.at[idx_v], rows_v, sem).wait()
        pltpu.sync_copy(rows_v, out_hbm.at[pl.ds(base, per)])

    return k


def _make_sc_combine(n_tok, d):
    info = plsc.get_sparse_core_info()
    nw = info.num_cores * info.num_subcores
    per_t = n_tok // nw
    per_r = _K * per_t
    mesh = plsc.VectorSubcoreMesh(core_axis_name="c", subcore_axis_name="s")

    @functools.partial(
        pl.kernel,
        out_type=jax.ShapeDtypeStruct((n_tok, d), jnp.float32),
        mesh=mesh,
        scratch_types=[
            pltpu.VMEM((per_r,), jnp.int32),
            pltpu.VMEM((per_r, d), jnp.float32),
            pltpu.SemaphoreType.DMA,
        ],
    )
    def k(h_hbm, inv_hbm, out_hbm, idx_v, rows_v, sem):
        wid = lax.axis_index("s") * info.num_cores + lax.axis_index("c")
        pltpu.sync_copy(inv_hbm.at[pl.ds(wid * per_r, per_r)], idx_v)
        pltpu.async_copy(h_hbm.at[idx_v], rows_v, sem).wait()

        def body(i, carry):
            for c in range(d // info.num_lanes):
                sl = pl.ds(c * info.num_lanes, info.num_lanes)
                rows_v[i, sl] = rows_v[2 * i, sl] + rows_v[2 * i + 1, sl]
            return carry

        lax.fori_loop(0, per_t, body, 0)
        pltpu.sync_copy(rows_v.at[pl.ds(0, per_t)],
                        out_hbm.at[pl.ds(wid * per_t, per_t)])

    return k


def _gelu_exact(a):
    return 0.5 * a * (1.0 + lax.erf(a * 0.7071067811865476))


def _ffn_body(tile_ref, exp_ref, start_ref, end_ref,
              xg_ref, fc_ref, proj_ref, gate_ref, h_ref):
    p = pl.program_id(0)
    t = tile_ref[p]
    gid = t * _TILE + lax.broadcasted_iota(jnp.int32, (_TILE, 1), 0)
    mask = (gid >= start_ref[p]) & (gid < end_ref[p])
    x = jnp.where(mask, xg_ref[...], 0.0).astype(jnp.bfloat16)
    a = lax.dot_general(x, fc_ref[0].astype(jnp.bfloat16),
                        (((1,), (1,)), ((), ())),
                        preferred_element_type=jnp.float32)
    g = _gelu_exact(a).astype(jnp.bfloat16)
    h = lax.dot_general(g, proj_ref[0].astype(jnp.bfloat16),
                        (((1,), (1,)), ((), ())),
                        preferred_element_type=jnp.float32)
    h = h * gate_ref[...]
    first = (p == 0) | (t != tile_ref[jnp.maximum(p - 1, 0)])

    @pl.when(first)
    def _():
        h_ref[...] = h

    @pl.when(jnp.logical_not(first))
    def _():
        h_ref[...] += h


def _run_ffn(pair_tile, pair_exp, pair_start, pair_end,
             xg, c_fc_weight, c_proj_weight, gates2d):
    n_rows, d = xg.shape
    e, dff, _ = c_fc_weight.shape
    np_ = pair_tile.shape[0]
    grid_spec = pltpu.PrefetchScalarGridSpec(
        num_scalar_prefetch=4,
        grid=(np_,),
        in_specs=[
            pl.BlockSpec((_TILE, d), lambda p, tr, er, sr, nr: (tr[p], 0)),
            pl.BlockSpec((1, dff, d), lambda p, tr, er, sr, nr: (er[p], 0, 0)),
            pl.BlockSpec((1, d, dff), lambda p, tr, er, sr, nr: (er[p], 0, 0)),
            pl.BlockSpec((_TILE, 1), lambda p, tr, er, sr, nr: (tr[p], 0)),
        ],
        out_specs=pl.BlockSpec((_TILE, d), lambda p, tr, er, sr, nr: (tr[p], 0)),
    )
    return pl.pallas_call(
        _ffn_body,
        grid_spec=grid_spec,
        out_shape=jax.ShapeDtypeStruct((n_rows, d), jnp.float32),
        compiler_params=pltpu.CompilerParams(
            dimension_semantics=("arbitrary",)),
    )(pair_tile, pair_exp, pair_start, pair_end,
      xg, c_fc_weight, c_proj_weight, gates2d)


def kernel(hidden_states, gate_weight, c_fc_weight, c_proj_weight):
    b, s, d = hidden_states.shape
    e, dff, _ = c_fc_weight.shape
    x = hidden_states.reshape(-1, d)
    n_tok = x.shape[0]
    n_rows = n_tok * _K
    nt = n_rows // _TILE

    idx8, w8 = _run_router(x, gate_weight)
    sel_flat = idx8[:, :_K].reshape(-1)
    gates_flat = w8[:, :_K].reshape(-1)

    perm = jnp.argsort(sel_flat, stable=True).astype(jnp.int32)
    batch_index = perm // _K
    gates_sorted = gates_flat[perm]
    sel_sorted = sel_flat[perm]
    interior = jnp.searchsorted(
        sel_sorted, jnp.arange(1, e, dtype=sel_sorted.dtype),
        side="left").astype(jnp.int32)
    tile_starts = jnp.arange(nt, dtype=jnp.int32) * _TILE
    bounds = jnp.sort(jnp.concatenate([tile_starts, interior]))
    pair_tile = jnp.clip(bounds // _TILE, 0, nt - 1)
    pair_exp = jnp.clip(
        jnp.searchsorted(interior, bounds, side="right").astype(jnp.int32),
        0, e - 1)
    pair_start = bounds
    pair_end = jnp.concatenate(
        [bounds[1:], jnp.array([n_rows], dtype=jnp.int32)])
    inv = jnp.zeros((n_rows,), jnp.int32).at[perm].set(
        jnp.arange(n_rows, dtype=jnp.int32))

    xg = _make_sc_gather(n_rows, d)(x, batch_index)
    h = _run_ffn(pair_tile, pair_exp, pair_start, pair_end,
                 xg, c_fc_weight, c_proj_weight,
                 gates_sorted.reshape(n_rows, 1))
    out = _make_sc_combine(n_tok, d)(h, inv)
    return out.reshape(b, s, d)


# speedup vs baseline: 1.5357x; 1.3242x over previous
import functools

import jax
import jax.numpy as jnp
from jax import lax
from jax.experimental import pallas as pl
from jax.experimental.pallas import tpu as pltpu
from jax.experimental.pallas import tpu_sc as plsc

_TILE = 128
_BLK = 128
_K = 2


def _router_body(x_ref, wg_ref, pos_ref, gw_ref, off_ref):
    x = x_ref[...]
    wg = wg_ref[...]
    logits = lax.dot_general(x, wg, (((1,), (1,)), ((), ())),
                             preferred_element_type=jnp.float32)
    s, e = logits.shape
    eio = lax.broadcasted_iota(jnp.int32, (s, e), 1)
    m1 = jnp.max(logits, axis=1, keepdims=True)
    i1 = jnp.min(jnp.where(logits == m1, eio, e), axis=1, keepdims=True)
    l2 = jnp.where(eio == i1, -jnp.inf, logits)
    m2 = jnp.max(l2, axis=1, keepdims=True)
    i2 = jnp.min(jnp.where(l2 == m2, eio, e), axis=1, keepdims=True)
    w1 = 1.0 / (1.0 + jnp.exp(m2 - m1))

    oh0 = (eio == i1).astype(jnp.float32)
    oh1 = (eio == i2).astype(jnp.float32)
    oh = oh0 + oh1

    rio = lax.broadcasted_iota(jnp.int32, (_BLK, _BLK), 0)
    cio = lax.broadcasted_iota(jnp.int32, (_BLK, _BLK), 1)
    tri = (cio < rio).astype(jnp.float32)
    nblk = s // _BLK
    c_blocks = []
    running = jnp.zeros((1, e), jnp.int32)
    for b in range(nblk):
        rb = oh[b * _BLK:(b + 1) * _BLK, :]
        cb = lax.dot_general(tri, rb, (((1,), (0,)), ((), ())),
                             preferred_element_type=jnp.float32)
        c_blocks.append(cb.astype(jnp.int32) + running)
        running = running + jnp.sum(rb, axis=0, keepdims=True).astype(jnp.int32)
    c = jnp.concatenate(c_blocks, axis=0)

    offs_cols = [jnp.zeros((1, 1), jnp.int32)]
    acc = jnp.zeros((1, 1), jnp.int32)
    for j in range(e - 1):
        acc = acc + running[:, j:j + 1]
        offs_cols.append(acc)
    offs = jnp.concatenate(offs_cols, axis=1)

    p = c + offs
    oh0i = oh0.astype(jnp.int32)
    oh1i = oh1.astype(jnp.int32)
    pos0 = jnp.sum(oh0i * p, axis=1, keepdims=True)
    pos1 = jnp.sum(oh1i * p, axis=1, keepdims=True)

    pos_ref[...] = jnp.where(eio == 0, pos0, jnp.where(eio == 1, pos1, 0))
    lio = lax.broadcasted_iota(jnp.int32, (s, 2 * 16), 1)
    gw_ref[...] = jnp.where(lio < 16, w1, 1.0 - w1)
    off_ref[...] = offs


def _run_router(x, gate_weight):
    s, _ = x.shape
    e = gate_weight.shape[0]
    return pl.pallas_call(
        _router_body,
        out_shape=(
            jax.ShapeDtypeStruct((s, e), jnp.int32),
            jax.ShapeDtypeStruct((s, 32), jnp.float32),
            jax.ShapeDtypeStruct((1, e), jnp.int32),
        ),
    )(x, gate_weight)


def _make_sc_dispatch(n_tok, d):
    info = plsc.get_sparse_core_info()
    nw = info.num_cores * info.num_subcores
    per = n_tok // nw
    mesh = plsc.VectorSubcoreMesh(core_axis_name="c", subcore_axis_name="s")

    @functools.partial(
        pl.kernel,
        out_type=jax.ShapeDtypeStruct((n_tok * _K, d), jnp.float32),
        mesh=mesh,
        scratch_types=[
            pltpu.VMEM((per,), jnp.int32),
            pltpu.VMEM((per,), jnp.int32),
            pltpu.VMEM((per, d), jnp.float32),
            pltpu.SemaphoreType.DMA,
        ],
    )
    def k(x_hbm, post_hbm, xg_hbm, idx0_v, idx1_v, xloc_v, sem):
        wid = lax.axis_index("s") * info.num_cores + lax.axis_index("c")
        base = wid * per
        pltpu.sync_copy(x_hbm.at[pl.ds(base, per)], xloc_v)
        pltpu.sync_copy(post_hbm.at[0, pl.ds(base, per)], idx0_v)
        pltpu.sync_copy(post_hbm.at[1, pl.ds(base, per)], idx1_v)
        cp0 = pltpu.async_copy(xloc_v, xg_hbm.at[idx0_v], sem)
        cp1 = pltpu.async_copy(xloc_v, xg_hbm.at[idx1_v], sem)
        cp0.wait()
        cp1.wait()

    return k


def _make_sc_combine(n_tok, d):
    info = plsc.get_sparse_core_info()
    nl = info.num_lanes
    nw = info.num_cores * info.num_subcores
    per = n_tok // nw
    mesh = plsc.VectorSubcoreMesh(core_axis_name="c", subcore_axis_name="s")

    @functools.partial(
        pl.kernel,
        out_type=jax.ShapeDtypeStruct((n_tok, d), jnp.float32),
        mesh=mesh,
        scratch_types=[
            pltpu.VMEM((per,), jnp.int32),
            pltpu.VMEM((per,), jnp.int32),
            pltpu.VMEM((per, 2 * nl), jnp.float32),
            pltpu.VMEM((per, d), jnp.float32),
            pltpu.VMEM((per, d), jnp.float32),
            pltpu.SemaphoreType.DMA,
        ],
    )
    def k(h_hbm, post_hbm, gw_hbm, out_hbm,
          idx0_v, idx1_v, gw_v, rows0_v, rows1_v, sem):
        wid = lax.axis_index("s") * info.num_cores + lax.axis_index("c")
        base = wid * per
        pltpu.sync_copy(post_hbm.at[0, pl.ds(base, per)], idx0_v)
        pltpu.sync_copy(post_hbm.at[1, pl.ds(base, per)], idx1_v)
        pltpu.sync_copy(gw_hbm.at[pl.ds(base, per)], gw_v)
        cp0 = pltpu.async_copy(h_hbm.at[idx0_v], rows0_v, sem)
        cp1 = pltpu.async_copy(h_hbm.at[idx1_v], rows1_v, sem)
        cp0.wait()
        cp1.wait()

        def body(i, carry):
            g0 = gw_v[i, pl.ds(0, nl)]
            g1 = gw_v[i, pl.ds(nl, nl)]
            for cch in range(d // nl):
                sl = pl.ds(cch * nl, nl)
                rows0_v[i, sl] = rows0_v[i, sl] * g0 + rows1_v[i, sl] * g1
            return carry

        lax.fori_loop(0, per, body, 0)
        pltpu.sync_copy(rows0_v, out_hbm.at[pl.ds(base, per)])

    return k


def _gelu_exact(a):
    return 0.5 * a * (1.0 + lax.erf(a * 0.7071067811865476))


def _ffn_body(tile_ref, exp_ref, start_ref, end_ref,
              xg_ref, fc_ref, proj_ref, h_ref):
    p = pl.program_id(0)
    t = tile_ref[p]
    gid = t * _TILE + lax.broadcasted_iota(jnp.int32, (_TILE, 1), 0)
    mask = (gid >= start_ref[p]) & (gid < end_ref[p])
    x = jnp.where(mask, xg_ref[...], 0.0).astype(jnp.bfloat16)
    a = lax.dot_general(x, fc_ref[0].astype(jnp.bfloat16),
                        (((1,), (1,)), ((), ())),
                        preferred_element_type=jnp.float32)
    g = _gelu_exact(a).astype(jnp.bfloat16)
    h = lax.dot_general(g, proj_ref[0].astype(jnp.bfloat16),
                        (((1,), (1,)), ((), ())),
                        preferred_element_type=jnp.float32)
    first = (p == 0) | (t != tile_ref[jnp.maximum(p - 1, 0)])

    @pl.when(first)
    def _():
        h_ref[...] = h

    @pl.when(jnp.logical_not(first))
    def _():
        h_ref[...] += h


def _run_ffn(pair_tile, pair_exp, pair_start, pair_end,
             xg, c_fc_weight, c_proj_weight):
    n_rows, d = xg.shape
    e, dff, _ = c_fc_weight.shape
    np_ = pair_tile.shape[0]
    grid_spec = pltpu.PrefetchScalarGridSpec(
        num_scalar_prefetch=4,
        grid=(np_,),
        in_specs=[
            pl.BlockSpec((_TILE, d), lambda p, tr, er, sr, nr: (tr[p], 0)),
            pl.BlockSpec((1, dff, d), lambda p, tr, er, sr, nr: (er[p], 0, 0)),
            pl.BlockSpec((1, d, dff), lambda p, tr, er, sr, nr: (er[p], 0, 0)),
        ],
        out_specs=pl.BlockSpec((_TILE, d), lambda p, tr, er, sr, nr: (tr[p], 0)),
    )
    return pl.pallas_call(
        _ffn_body,
        grid_spec=grid_spec,
        out_shape=jax.ShapeDtypeStruct((n_rows, d), jnp.float32),
        compiler_params=pltpu.CompilerParams(
            dimension_semantics=("arbitrary",)),
    )(pair_tile, pair_exp, pair_start, pair_end,
      xg, c_fc_weight, c_proj_weight)


def kernel(hidden_states, gate_weight, c_fc_weight, c_proj_weight):
    b, s, d = hidden_states.shape
    e, dff, _ = c_fc_weight.shape
    x = hidden_states.reshape(-1, d)
    n_tok = x.shape[0]
    n_rows = n_tok * _K
    nt = n_rows // _TILE

    pos8, gw, offs = _run_router(x, gate_weight)
    post = pos8[:, :_K].T

    interior = offs[0, 1:]
    tile_starts = jnp.arange(nt, dtype=jnp.int32) * _TILE
    bounds = jnp.sort(jnp.concatenate([tile_starts, interior]))
    pair_tile = jnp.clip(bounds // _TILE, 0, nt - 1)
    pair_exp = jnp.clip(
        jnp.searchsorted(interior, bounds, side="right").astype(jnp.int32),
        0, e - 1)
    pair_start = bounds
    pair_end = jnp.concatenate(
        [bounds[1:], jnp.array([n_rows], dtype=jnp.int32)])

    xg = _make_sc_dispatch(n_tok, d)(x, post)
    h = _run_ffn(pair_tile, pair_exp, pair_start, pair_end,
                 xg, c_fc_weight, c_proj_weight)
    out = _make_sc_combine(n_tok, d)(h, post, gw)
    return out.reshape(b, s, d)


# speedup vs baseline: 2.1099x; 1.3739x over previous
import functools

import jax
import jax.numpy as jnp
from jax import lax
from jax.experimental import pallas as pl
from jax.experimental.pallas import tpu as pltpu
from jax.experimental.pallas import tpu_sc as plsc

_TILE = 256
_BLK = 128
_K = 2


def _router_body(x_ref, wg_ref, pos_ref, gw_ref, off_ref):
    x = x_ref[...]
    wg = wg_ref[...]
    logits = lax.dot_general(x, wg, (((1,), (1,)), ((), ())),
                             preferred_element_type=jnp.float32)
    s, e = logits.shape
    eio = lax.broadcasted_iota(jnp.int32, (s, e), 1)
    m1 = jnp.max(logits, axis=1, keepdims=True)
    i1 = jnp.min(jnp.where(logits == m1, eio, e), axis=1, keepdims=True)
    l2 = jnp.where(eio == i1, -jnp.inf, logits)
    m2 = jnp.max(l2, axis=1, keepdims=True)
    i2 = jnp.min(jnp.where(l2 == m2, eio, e), axis=1, keepdims=True)
    w1 = 1.0 / (1.0 + jnp.exp(m2 - m1))

    oh0 = (eio == i1).astype(jnp.float32)
    oh1 = (eio == i2).astype(jnp.float32)
    oh = oh0 + oh1

    rio = lax.broadcasted_iota(jnp.int32, (_BLK, _BLK), 0)
    cio = lax.broadcasted_iota(jnp.int32, (_BLK, _BLK), 1)
    tri = (cio < rio).astype(jnp.float32)
    nblk = s // _BLK
    c_blocks = []
    running = jnp.zeros((1, e), jnp.int32)
    for b in range(nblk):
        rb = oh[b * _BLK:(b + 1) * _BLK, :]
        cb = lax.dot_general(tri, rb, (((1,), (0,)), ((), ())),
                             preferred_element_type=jnp.float32)
        c_blocks.append(cb.astype(jnp.int32) + running)
        running = running + jnp.sum(rb, axis=0, keepdims=True).astype(jnp.int32)
    c = jnp.concatenate(c_blocks, axis=0)

    offs_cols = [jnp.zeros((1, 1), jnp.int32)]
    acc = jnp.zeros((1, 1), jnp.int32)
    for j in range(e - 1):
        acc = acc + running[:, j:j + 1]
        offs_cols.append(acc)
    offs = jnp.concatenate(offs_cols, axis=1)

    p = c + offs
    oh0i = oh0.astype(jnp.int32)
    oh1i = oh1.astype(jnp.int32)
    pos0 = jnp.sum(oh0i * p, axis=1, keepdims=True)
    pos1 = jnp.sum(oh1i * p, axis=1, keepdims=True)

    pos_ref[...] = jnp.where(eio == 0, pos0, jnp.where(eio == 1, pos1, 0))
    lio = lax.broadcasted_iota(jnp.int32, (s, 2 * 16), 1)
    gw_ref[...] = jnp.where(lio < 16, w1, 1.0 - w1)
    off_ref[...] = offs


def _run_router(x, gate_weight):
    s, _ = x.shape
    e = gate_weight.shape[0]
    return pl.pallas_call(
        _router_body,
        out_shape=(
            jax.ShapeDtypeStruct((s, e), jnp.int32),
            jax.ShapeDtypeStruct((s, 32), jnp.float32),
            jax.ShapeDtypeStruct((1, e), jnp.int32),
        ),
    )(x, gate_weight)


def _make_sc_dispatch(n_tok, d):
    info = plsc.get_sparse_core_info()
    nw = info.num_cores * info.num_subcores
    per = n_tok // nw
    mesh = plsc.VectorSubcoreMesh(core_axis_name="c", subcore_axis_name="s")

    @functools.partial(
        pl.kernel,
        out_type=jax.ShapeDtypeStruct((n_tok * _K, d), jnp.float32),
        mesh=mesh,
        scratch_types=[
            pltpu.VMEM((per,), jnp.int32),
            pltpu.VMEM((per,), jnp.int32),
            pltpu.VMEM((per, d), jnp.float32),
            pltpu.SemaphoreType.DMA,
        ],
    )
    def k(x_hbm, post_hbm, xg_hbm, idx0_v, idx1_v, xloc_v, sem):
        wid = lax.axis_index("s") * info.num_cores + lax.axis_index("c")
        base = wid * per
        pltpu.sync_copy(x_hbm.at[pl.ds(base, per)], xloc_v)
        pltpu.sync_copy(post_hbm.at[0, pl.ds(base, per)], idx0_v)
        pltpu.sync_copy(post_hbm.at[1, pl.ds(base, per)], idx1_v)
        cp0 = pltpu.async_copy(xloc_v, xg_hbm.at[idx0_v], sem)
        cp1 = pltpu.async_copy(xloc_v, xg_hbm.at[idx1_v], sem)
        cp0.wait()
        cp1.wait()

    return k


def _make_sc_combine(n_tok, d):
    info = plsc.get_sparse_core_info()
    nl = info.num_lanes
    nw = info.num_cores * info.num_subcores
    per = n_tok // nw
    mesh = plsc.VectorSubcoreMesh(core_axis_name="c", subcore_axis_name="s")

    @functools.partial(
        pl.kernel,
        out_type=jax.ShapeDtypeStruct((n_tok, d), jnp.float32),
        mesh=mesh,
        scratch_types=[
            pltpu.VMEM((per,), jnp.int32),
            pltpu.VMEM((per,), jnp.int32),
            pltpu.VMEM((per, 2 * nl), jnp.float32),
            pltpu.VMEM((per, d), jnp.float32),
            pltpu.VMEM((per, d), jnp.float32),
            pltpu.SemaphoreType.DMA,
        ],
    )
    def k(h_hbm, post_hbm, gw_hbm, out_hbm,
          idx0_v, idx1_v, gw_v, rows0_v, rows1_v, sem):
        wid = lax.axis_index("s") * info.num_cores + lax.axis_index("c")
        base = wid * per
        pltpu.sync_copy(post_hbm.at[0, pl.ds(base, per)], idx0_v)
        pltpu.sync_copy(post_hbm.at[1, pl.ds(base, per)], idx1_v)
        pltpu.sync_copy(gw_hbm.at[pl.ds(base, per)], gw_v)
        cp0 = pltpu.async_copy(h_hbm.at[idx0_v], rows0_v, sem)
        cp1 = pltpu.async_copy(h_hbm.at[idx1_v], rows1_v, sem)
        cp0.wait()
        cp1.wait()

        def body(i, carry):
            g0 = gw_v[i, pl.ds(0, nl)]
            g1 = gw_v[i, pl.ds(nl, nl)]
            for cch in range(d // nl):
                sl = pl.ds(cch * nl, nl)
                rows0_v[i, sl] = rows0_v[i, sl] * g0 + rows1_v[i, sl] * g1
            return carry

        lax.fori_loop(0, per, body, 0)
        pltpu.sync_copy(rows0_v, out_hbm.at[pl.ds(base, per)])

    return k


def _gelu_exact(a):
    return 0.5 * a * (1.0 + lax.erf(a * 0.7071067811865476))


def _ffn_body(tile_ref, exp_ref, start_ref, end_ref,
              xg_ref, fc_ref, proj_ref, h_ref):
    p = pl.program_id(0)
    t = tile_ref[p]
    gid = t * _TILE + lax.broadcasted_iota(jnp.int32, (_TILE, 1), 0)
    mask = (gid >= start_ref[p]) & (gid < end_ref[p])
    x = jnp.where(mask, xg_ref[...], 0.0).astype(jnp.bfloat16)
    a = lax.dot_general(x, fc_ref[0].astype(jnp.bfloat16),
                        (((1,), (1,)), ((), ())),
                        preferred_element_type=jnp.float32)
    g = _gelu_exact(a).astype(jnp.bfloat16)
    h = lax.dot_general(g, proj_ref[0].astype(jnp.bfloat16),
                        (((1,), (1,)), ((), ())),
                        preferred_element_type=jnp.float32)
    first = (p == 0) | (t != tile_ref[jnp.maximum(p - 1, 0)])

    @pl.when(first)
    def _():
        h_ref[...] = h

    @pl.when(jnp.logical_not(first))
    def _():
        h_ref[...] += h


def _run_ffn(pair_tile, pair_exp, pair_start, pair_end,
             xg, c_fc_weight, c_proj_weight):
    n_rows, d = xg.shape
    e, dff, _ = c_fc_weight.shape
    np_ = pair_tile.shape[0]
    grid_spec = pltpu.PrefetchScalarGridSpec(
        num_scalar_prefetch=4,
        grid=(np_,),
        in_specs=[
            pl.BlockSpec((_TILE, d), lambda p, tr, er, sr, nr: (tr[p], 0)),
            pl.BlockSpec((1, dff, d), lambda p, tr, er, sr, nr: (er[p], 0, 0)),
            pl.BlockSpec((1, d, dff), lambda p, tr, er, sr, nr: (er[p], 0, 0)),
        ],
        out_specs=pl.BlockSpec((_TILE, d), lambda p, tr, er, sr, nr: (tr[p], 0)),
    )
    return pl.pallas_call(
        _ffn_body,
        grid_spec=grid_spec,
        out_shape=jax.ShapeDtypeStruct((n_rows, d), jnp.float32),
        compiler_params=pltpu.CompilerParams(
            dimension_semantics=("arbitrary",)),
    )(pair_tile, pair_exp, pair_start, pair_end,
      xg, c_fc_weight, c_proj_weight)


def kernel(hidden_states, gate_weight, c_fc_weight, c_proj_weight):
    b, s, d = hidden_states.shape
    e, dff, _ = c_fc_weight.shape
    x = hidden_states.reshape(-1, d)
    n_tok = x.shape[0]
    n_rows = n_tok * _K
    nt = n_rows // _TILE

    pos8, gw, offs = _run_router(x, gate_weight)
    post = pos8[:, :_K].T

    interior = offs[0, 1:]
    tile_starts = jnp.arange(nt, dtype=jnp.int32) * _TILE
    bounds = jnp.sort(jnp.concatenate([tile_starts, interior]))
    pair_tile = jnp.clip(bounds // _TILE, 0, nt - 1)
    pair_exp = jnp.clip(
        jnp.searchsorted(interior, bounds, side="right").astype(jnp.int32),
        0, e - 1)
    pair_start = bounds
    pair_end = jnp.concatenate(
        [bounds[1:], jnp.array([n_rows], dtype=jnp.int32)])

    xg = _make_sc_dispatch(n_tok, d)(x, post)
    h = _run_ffn(pair_tile, pair_exp, pair_start, pair_end,
                 xg, c_fc_weight, c_proj_weight)
    out = _make_sc_combine(n_tok, d)(h, post, gw)
    return out.reshape(b, s, d)


# speedup vs baseline: 2.1985x; 1.0420x over previous
import functools

import jax
import jax.numpy as jnp
from jax import lax
from jax.experimental import pallas as pl
from jax.experimental.pallas import tpu as pltpu
from jax.experimental.pallas import tpu_sc as plsc

_TILE = 512
_BLK = 128
_K = 2


def _router_body(x_ref, wg_ref, pos_ref, gw_ref, off_ref):
    x = x_ref[...]
    wg = wg_ref[...]
    logits = lax.dot_general(x, wg, (((1,), (1,)), ((), ())),
                             preferred_element_type=jnp.float32)
    s, e = logits.shape
    eio = lax.broadcasted_iota(jnp.int32, (s, e), 1)
    m1 = jnp.max(logits, axis=1, keepdims=True)
    i1 = jnp.min(jnp.where(logits == m1, eio, e), axis=1, keepdims=True)
    l2 = jnp.where(eio == i1, -jnp.inf, logits)
    m2 = jnp.max(l2, axis=1, keepdims=True)
    i2 = jnp.min(jnp.where(l2 == m2, eio, e), axis=1, keepdims=True)
    w1 = 1.0 / (1.0 + jnp.exp(m2 - m1))

    oh0 = (eio == i1).astype(jnp.float32)
    oh1 = (eio == i2).astype(jnp.float32)
    oh = oh0 + oh1

    rio = lax.broadcasted_iota(jnp.int32, (_BLK, _BLK), 0)
    cio = lax.broadcasted_iota(jnp.int32, (_BLK, _BLK), 1)
    tri = (cio < rio).astype(jnp.float32)
    nblk = s // _BLK
    c_blocks = []
    running = jnp.zeros((1, e), jnp.int32)
    for b in range(nblk):
        rb = oh[b * _BLK:(b + 1) * _BLK, :]
        cb = lax.dot_general(tri, rb, (((1,), (0,)), ((), ())),
                             preferred_element_type=jnp.float32)
        c_blocks.append(cb.astype(jnp.int32) + running)
        running = running + jnp.sum(rb, axis=0, keepdims=True).astype(jnp.int32)
    c = jnp.concatenate(c_blocks, axis=0)

    offs_cols = [jnp.zeros((1, 1), jnp.int32)]
    acc = jnp.zeros((1, 1), jnp.int32)
    for j in range(e - 1):
        acc = acc + running[:, j:j + 1]
        offs_cols.append(acc)
    offs = jnp.concatenate(offs_cols, axis=1)

    p = c + offs
    oh0i = oh0.astype(jnp.int32)
    oh1i = oh1.astype(jnp.int32)
    pos0 = jnp.sum(oh0i * p, axis=1, keepdims=True)
    pos1 = jnp.sum(oh1i * p, axis=1, keepdims=True)

    pos_ref[...] = jnp.where(eio == 0, pos0, jnp.where(eio == 1, pos1, 0))
    lio = lax.broadcasted_iota(jnp.int32, (s, 2 * 16), 1)
    gw_ref[...] = jnp.where(lio < 16, w1, 1.0 - w1)
    off_ref[...] = offs


def _run_router(x, gate_weight):
    s, _ = x.shape
    e = gate_weight.shape[0]
    return pl.pallas_call(
        _router_body,
        out_shape=(
            jax.ShapeDtypeStruct((s, e), jnp.int32),
            jax.ShapeDtypeStruct((s, 32), jnp.float32),
            jax.ShapeDtypeStruct((1, e), jnp.int32),
        ),
    )(x, gate_weight)


def _make_sc_dispatch(n_tok, d):
    info = plsc.get_sparse_core_info()
    nw = info.num_cores * info.num_subcores
    per = n_tok // nw
    mesh = plsc.VectorSubcoreMesh(core_axis_name="c", subcore_axis_name="s")

    @functools.partial(
        pl.kernel,
        out_type=jax.ShapeDtypeStruct((n_tok * _K, d), jnp.float32),
        mesh=mesh,
        scratch_types=[
            pltpu.VMEM((per,), jnp.int32),
            pltpu.VMEM((per,), jnp.int32),
            pltpu.VMEM((per, d), jnp.float32),
            pltpu.SemaphoreType.DMA,
        ],
    )
    def k(x_hbm, post_hbm, xg_hbm, idx0_v, idx1_v, xloc_v, sem):
        wid = lax.axis_index("s") * info.num_cores + lax.axis_index("c")
        base = wid * per
        pltpu.sync_copy(x_hbm.at[pl.ds(base, per)], xloc_v)
        pltpu.sync_copy(post_hbm.at[0, pl.ds(base, per)], idx0_v)
        pltpu.sync_copy(post_hbm.at[1, pl.ds(base, per)], idx1_v)
        cp0 = pltpu.async_copy(xloc_v, xg_hbm.at[idx0_v], sem)
        cp1 = pltpu.async_copy(xloc_v, xg_hbm.at[idx1_v], sem)
        cp0.wait()
        cp1.wait()

    return k


def _make_sc_combine(n_tok, d):
    info = plsc.get_sparse_core_info()
    nl = info.num_lanes
    nw = info.num_cores * info.num_subcores
    per = n_tok // nw
    mesh = plsc.VectorSubcoreMesh(core_axis_name="c", subcore_axis_name="s")

    @functools.partial(
        pl.kernel,
        out_type=jax.ShapeDtypeStruct((n_tok, d), jnp.float32),
        mesh=mesh,
        scratch_types=[
            pltpu.VMEM((per,), jnp.int32),
            pltpu.VMEM((per,), jnp.int32),
            pltpu.VMEM((per, 2 * nl), jnp.float32),
            pltpu.VMEM((per, d), jnp.float32),
            pltpu.VMEM((per, d), jnp.float32),
            pltpu.SemaphoreType.DMA,
        ],
    )
    def k(h_hbm, post_hbm, gw_hbm, out_hbm,
          idx0_v, idx1_v, gw_v, rows0_v, rows1_v, sem):
        wid = lax.axis_index("s") * info.num_cores + lax.axis_index("c")
        base = wid * per
        pltpu.sync_copy(post_hbm.at[0, pl.ds(base, per)], idx0_v)
        pltpu.sync_copy(post_hbm.at[1, pl.ds(base, per)], idx1_v)
        pltpu.sync_copy(gw_hbm.at[pl.ds(base, per)], gw_v)
        cp0 = pltpu.async_copy(h_hbm.at[idx0_v], rows0_v, sem)
        cp1 = pltpu.async_copy(h_hbm.at[idx1_v], rows1_v, sem)
        cp0.wait()
        cp1.wait()

        def body(i, carry):
            g0 = gw_v[i, pl.ds(0, nl)]
            g1 = gw_v[i, pl.ds(nl, nl)]
            for cch in range(d // nl):
                sl = pl.ds(cch * nl, nl)
                rows0_v[i, sl] = rows0_v[i, sl] * g0 + rows1_v[i, sl] * g1
            return carry

        lax.fori_loop(0, per, body, 0)
        pltpu.sync_copy(rows0_v, out_hbm.at[pl.ds(base, per)])

    return k


def _gelu_exact(a):
    return 0.5 * a * (1.0 + lax.erf(a * 0.7071067811865476))


def _ffn_body(tile_ref, exp_ref, start_ref, end_ref,
              xg_ref, fc_ref, proj_ref, h_ref):
    p = pl.program_id(0)
    t = tile_ref[p]
    gid = t * _TILE + lax.broadcasted_iota(jnp.int32, (_TILE, 1), 0)
    mask = (gid >= start_ref[p]) & (gid < end_ref[p])
    x = jnp.where(mask, xg_ref[...], 0.0).astype(jnp.bfloat16)
    a = lax.dot_general(x, fc_ref[0].astype(jnp.bfloat16),
                        (((1,), (1,)), ((), ())),
                        preferred_element_type=jnp.float32)
    g = _gelu_exact(a).astype(jnp.bfloat16)
    h = lax.dot_general(g, proj_ref[0].astype(jnp.bfloat16),
                        (((1,), (1,)), ((), ())),
                        preferred_element_type=jnp.float32)
    first = (p == 0) | (t != tile_ref[jnp.maximum(p - 1, 0)])

    @pl.when(first)
    def _():
        h_ref[...] = h

    @pl.when(jnp.logical_not(first))
    def _():
        h_ref[...] += h


def _run_ffn(pair_tile, pair_exp, pair_start, pair_end,
             xg, c_fc_weight, c_proj_weight):
    n_rows, d = xg.shape
    e, dff, _ = c_fc_weight.shape
    np_ = pair_tile.shape[0]
    grid_spec = pltpu.PrefetchScalarGridSpec(
        num_scalar_prefetch=4,
        grid=(np_,),
        in_specs=[
            pl.BlockSpec((_TILE, d), lambda p, tr, er, sr, nr: (tr[p], 0)),
            pl.BlockSpec((1, dff, d), lambda p, tr, er, sr, nr: (er[p], 0, 0)),
            pl.BlockSpec((1, d, dff), lambda p, tr, er, sr, nr: (er[p], 0, 0)),
        ],
        out_specs=pl.BlockSpec((_TILE, d), lambda p, tr, er, sr, nr: (tr[p], 0)),
    )
    return pl.pallas_call(
        _ffn_body,
        grid_spec=grid_spec,
        out_shape=jax.ShapeDtypeStruct((n_rows, d), jnp.float32),
        compiler_params=pltpu.CompilerParams(
            dimension_semantics=("arbitrary",)),
    )(pair_tile, pair_exp, pair_start, pair_end,
      xg, c_fc_weight, c_proj_weight)


def kernel(hidden_states, gate_weight, c_fc_weight, c_proj_weight):
    b, s, d = hidden_states.shape
    e, dff, _ = c_fc_weight.shape
    x = hidden_states.reshape(-1, d)
    n_tok = x.shape[0]
    n_rows = n_tok * _K
    nt = n_rows // _TILE

    pos8, gw, offs = _run_router(x, gate_weight)
    post = pos8[:, :_K].T

    interior = offs[0, 1:]
    tile_starts = jnp.arange(nt, dtype=jnp.int32) * _TILE
    bounds = jnp.sort(jnp.concatenate([tile_starts, interior]))
    pair_tile = jnp.clip(bounds // _TILE, 0, nt - 1)
    pair_exp = jnp.clip(
        jnp.searchsorted(interior, bounds, side="right").astype(jnp.int32),
        0, e - 1)
    pair_start = bounds
    pair_end = jnp.concatenate(
        [bounds[1:], jnp.array([n_rows], dtype=jnp.int32)])

    xg = _make_sc_dispatch(n_tok, d)(x, post)
    h = _run_ffn(pair_tile, pair_exp, pair_start, pair_end,
                 xg, c_fc_weight, c_proj_weight)
    out = _make_sc_combine(n_tok, d)(h, post, gw)
    return out.reshape(b, s, d)
